# baseline (speedup 1.0000x reference)
import functools

import jax
import jax.numpy as jnp
from jax import lax
from jax.experimental import pallas as pl
from jax.experimental.pallas import tpu as pltpu
from jax.experimental.pallas import tpu_sc as plsc

EMBED_DIM = 64
MARGIN = 0.1
BN_EPS = 1e-5
B = 16384

NC, NS = 2, 16
NW = NC * NS
ROWS_W = B // NW
CHUNK = 128
NCH = ROWS_W // CHUNK

N_CE = 9
N_RE = 2
N_SLOT = N_CE + N_RE
N_CR = 8

BLK = 2048
NBLK = B // BLK


def _sc_gather(ce_pack, re_pack, cr_flat, idx_pair_w, idx_cr_w):
    mesh = plsc.VectorSubcoreMesh(core_axis_name="c", subcore_axis_name="s")

    @functools.partial(
        pl.kernel,
        out_type=(
            jax.ShapeDtypeStruct((N_SLOT * B, CHUNK), jnp.float32),
            jax.ShapeDtypeStruct((N_CR * B,), jnp.float32),
        ),
        mesh=mesh,
        scratch_types=[
            pltpu.VMEM((N_SLOT * ROWS_W,), jnp.int32),
            pltpu.VMEM((N_CR * ROWS_W,), jnp.int32),
            pltpu.VMEM((ROWS_W, CHUNK), jnp.float32),
            pltpu.VMEM((ROWS_W,), jnp.float32),
            pltpu.SemaphoreType.DMA,
        ],
    )
    def k(ce_hbm, re_hbm, cr_hbm, ipair_hbm, icr_hbm,
          big_out, rad_out, ip_v, ic_v, rows_v, rad_v, sem):
        wid = lax.axis_index("s") * NC + lax.axis_index("c")
        rbase = pl.multiple_of(wid * ROWS_W, ROWS_W)

        pltpu.sync_copy(
            ipair_hbm.at[pl.ds(pl.multiple_of(wid * (N_SLOT * ROWS_W),
                                              N_SLOT * ROWS_W),
                               N_SLOT * ROWS_W)], ip_v)
        pltpu.sync_copy(
            icr_hbm.at[pl.ds(pl.multiple_of(wid * (N_CR * ROWS_W),
                                            N_CR * ROWS_W),
                             N_CR * ROWS_W)], ic_v)

        def row_gather(table, s):
            cps = [
                pltpu.async_copy(
                    table.at[ip_v.at[pl.ds(
                        pl.multiple_of(s * ROWS_W + j * CHUNK, CHUNK),
                        CHUNK)]],
                    rows_v.at[pl.ds(j * CHUNK, CHUNK)],
                    sem,
                )
                for j in range(NCH)
            ]
            for c in cps:
                c.wait()
            pltpu.sync_copy(
                rows_v,
                big_out.at[pl.ds(pl.multiple_of(s * B + rbase, ROWS_W),
                                 ROWS_W), :])

        def ce_body(s, carry):
            row_gather(ce_hbm, s)
            return carry

        lax.fori_loop(0, N_CE, ce_body, 0, unroll=False)

        for t in range(N_RE):
            row_gather(re_hbm, N_CE + t)

        def rad_body(g, carry):
            cps = [
                pltpu.async_copy(
                    cr_hbm.at[ic_v.at[pl.ds(
                        pl.multiple_of(g * ROWS_W + j * CHUNK, CHUNK),
                        CHUNK)]],
                    rad_v.at[pl.ds(j * CHUNK, CHUNK)],
                    sem,
                )
                for j in range(NCH)
            ]
            for c in cps:
                c.wait()
            pltpu.sync_copy(
                rad_v,
                rad_out.at[pl.ds(pl.multiple_of(g * B + rbase, ROWS_W),
                                 ROWS_W)])
            return carry

        lax.fori_loop(0, N_CR, rad_body, 0, unroll=False)

    return k(ce_pack, re_pack, cr_flat, idx_pair_w, idx_cr_w)


def _tc_loss_body(big_ref, par_ref, rad_ref, gam_ref, bet_ref, out_ref,
                  ssum_ref, ssq_ref, acc_ref):
    p = pl.program_id(0)
    i = pl.program_id(1)

    x = big_ref[...]
    par = par_ref[...][:, :, None]
    xs = jnp.where(par == 1, x[:, :, EMBED_DIM:], x[:, :, :EMBED_DIM])

    @pl.when(jnp.logical_and(p == 0, i == 0))
    def _init():
        ssum_ref[...] = jnp.zeros_like(ssum_ref)
        ssq_ref[...] = jnp.zeros_like(ssq_ref)
        acc_ref[0, 0] = 0.0

    @pl.when(p == 0)
    def _stats():
        ssum_ref[...] += jnp.sum(xs, axis=1)
        ssq_ref[...] += jnp.sum(xs * xs, axis=1)

    @pl.when(p == 1)
    def _loss():
        gamma = gam_ref[...].reshape(1, EMBED_DIM)
        beta = bet_ref[...].reshape(1, EMBED_DIM)
        mean = ssum_ref[...] * (1.0 / B)
        var = ssq_ref[...] * (1.0 / B) - mean * mean
        rstd = lax.rsqrt(var + BN_EPS)

        def bn(s):
            return ((xs[s] - mean[s][None, :]) * rstd[s][None, :]
                    * gamma + beta)

        ra = jnp.abs(rad_ref[...])

        def norm(v):
            return jnp.sqrt(jnp.sum(v * v, axis=1) + 1e-12)

        relu = jax.nn.relu

        y0a, y0b = bn(0), bn(1)
        t = relu(norm(y0a - y0b) + ra[0] - ra[1] - MARGIN)

        y1a, y1b, y1c = bn(2), bn(3), bn(4)
        r1a, r1b = ra[2], ra[3]
        t += (relu(norm(y1b - y1a) - (r1a + r1b) - MARGIN)
              + relu(norm(y1c - y1a) - r1a - MARGIN)
              + relu(norm(y1c - y1b) - r1b - MARGIN))

        y2a, y2c = bn(5), bn(6)
        r2a, r2c = ra[4], ra[5]
        dst = norm(y2a + xs[9] - y2c)
        t += (relu(dst + r2a - r2c - MARGIN)
              + relu(r2a + r2c - dst + MARGIN))

        y3b, y3c = bn(7), bn(8)
        t += relu(norm(y3b - xs[10] - y3c) - ra[6] - ra[7] - MARGIN)

        acc_ref[0, 0] += jnp.sum(t)

    @pl.when(jnp.logical_and(p == 1, i == NBLK - 1))
    def _fin():
        out_ref[0, 0] = acc_ref[0, 0] * (1.0 / B)


def kernel(gci0, gci1, gci2, gci3, class_embed, class_rad, rel_embed, bn_gamma, bn_beta):
    idx_slot = jnp.stack([
        gci0[:, 0], gci0[:, 1],
        gci1[:, 0], gci1[:, 1], gci1[:, 2],
        gci2[:, 0], gci2[:, 2],
        gci3[:, 1], gci3[:, 2],
        gci2[:, 1], gci3[:, 0],
    ])
    idx_cr = jnp.stack([
        gci0[:, 0], gci0[:, 1],
        gci1[:, 0], gci1[:, 1],
        gci2[:, 0], gci2[:, 2],
        gci3[:, 1], gci3[:, 2],
    ])
    idx_pair = idx_slot >> 1
    parity = (idx_slot & 1).astype(jnp.int32)

    ipw = idx_pair.reshape(N_SLOT, NW, ROWS_W).transpose(1, 0, 2).reshape(-1)
    icw = idx_cr.reshape(N_CR, NW, ROWS_W).transpose(1, 0, 2).reshape(-1)

    ce_pack = class_embed.reshape(500000, 2 * EMBED_DIM)
    re_pack = rel_embed.reshape(500, 2 * EMBED_DIM)
    cr_flat = class_rad.reshape(-1)

    big, rad = _sc_gather(ce_pack, re_pack, cr_flat, ipw, icw)

    loss = pl.pallas_call(
        _tc_loss_body,
        out_shape=jax.ShapeDtypeStruct((1, 1), jnp.float32),
        grid=(2, NBLK),
        in_specs=[
            pl.BlockSpec((N_SLOT, BLK, CHUNK), lambda p, i: (0, i, 0)),
            pl.BlockSpec((N_SLOT, BLK), lambda p, i: (0, i)),
            pl.BlockSpec((N_CR, BLK), lambda p, i: (0, i)),
            pl.BlockSpec((1, EMBED_DIM), lambda p, i: (0, 0)),
            pl.BlockSpec((1, EMBED_DIM), lambda p, i: (0, 0)),
        ],
        out_specs=pl.BlockSpec((1, 1), lambda p, i: (0, 0),
                               memory_space=pltpu.SMEM),
        scratch_shapes=[
            pltpu.VMEM((N_SLOT, EMBED_DIM), jnp.float32),
            pltpu.VMEM((N_SLOT, EMBED_DIM), jnp.float32),
            pltpu.SMEM((1, 1), jnp.float32),
        ],
        compiler_params=pltpu.CompilerParams(
            vmem_limit_bytes=128 * 1024 * 1024),
    )(big.reshape(N_SLOT, B, CHUNK), parity, rad.reshape(N_CR, B),
      bn_gamma.reshape(1, EMBED_DIM), bn_beta.reshape(1, EMBED_DIM))

    return jnp.reshape(loss, ())
